# Initial kernel scaffold; baseline (speedup 1.0000x reference)
import jax
import jax.numpy as jnp
from jax.experimental import pallas as pl


def kernel(psi, alpha, scale_w, potential):
    raise NotImplementedError("write your pallas kernel here")


# speedup vs baseline: 4.5036x; 4.5036x over previous
import jax
import jax.numpy as jnp
from jax.experimental import pallas as pl
from jax.experimental.pallas import tpu as pltpu

_DT = 0.1
_HALF_DT = _DT / 2.0
_CG_MAX_ITER = 20
_CG_TOL = 1e-06
_DILS = (5, 10, 20)
_TILE = 256


def _ham(v, w, pot_eff):
    out = pot_eff * v
    for s, d in enumerate(_DILS):
        out = out - w[s] * (jnp.roll(v, d, axis=1) + jnp.roll(v, -d, axis=1))
    return out


def _body(scale_ref, psir_ref, psii_ref, alpha_ref, pot_ref, outr_ref, outi_ref):
    T = psir_ref.shape[0]
    pr = psir_ref[:]
    pi_ = psii_ref[:]
    inten = pr * pr + pi_ * pi_
    m = jnp.mean(inten, axis=1, keepdims=True)
    inten = inten / (m + 1e-08)
    phase = alpha_ref[:] * inten
    c = jnp.cos(phase)
    sn = jnp.sin(phase)
    rot_r = pr * c - pi_ * sn
    rot_i = pr * sn + pi_ * c

    w = (scale_ref[0], scale_ref[1], scale_ref[2])
    pot = pot_ref[:]

    v = jnp.concatenate([rot_r, rot_i], axis=0)
    Hv = _ham(v, w, pot)
    rhs = v + _HALF_DT * jnp.concatenate([Hv[T:], -Hv[:T]], axis=0)

    def a_minus(u):
        Hu = _ham(u, w, pot)
        return u + _HALF_DT * jnp.concatenate([-Hu[T:], Hu[:T]], axis=0)

    def tokdot(a, b):
        s = jnp.sum(a * b, axis=1, keepdims=True)
        return s[:T] + s[T:]

    r = rhs
    p = r
    x = jnp.zeros_like(r)
    rs_old = tokdot(r, r)

    def cg_iter(i, carry):
        x, r, p, rs_old = carry
        active = jnp.sqrt(rs_old) > _CG_TOL
        Ap = a_minus(p)
        pAp = tokdot(p, Ap)
        a = jnp.where(active, rs_old / (pAp + 1e-12), 0.0)
        a2 = jnp.concatenate([a, a], axis=0)
        x = x + a2 * p
        r = r - a2 * Ap
        rs_new = tokdot(r, r)
        beta = jnp.where(active, rs_new / (rs_old + 1e-12), 0.0)
        b2 = jnp.concatenate([beta, beta], axis=0)
        p = r + b2 * p
        rs_old = jnp.where(active, rs_new, rs_old)
        return (x, r, p, rs_old)

    x, r, p, rs_old = jax.lax.fori_loop(0, _CG_MAX_ITER, cg_iter, (x, r, p, rs_old))
    outr_ref[:] = x[:T]
    outi_ref[:] = x[T:]


def kernel(psi, alpha, scale_w, potential):
    b, s, d, _ = psi.shape
    n = b * s
    psir = psi[..., 0].reshape(n, d)
    psii = psi[..., 1].reshape(n, d)
    alpha2 = alpha.reshape(1, d)
    pot_eff = (potential + 2.0 * jnp.sum(scale_w)).reshape(1, d)

    grid = (n // _TILE,)
    out_r, out_i = pl.pallas_call(
        _body,
        grid=grid,
        in_specs=[
            pl.BlockSpec(memory_space=pltpu.SMEM),
            pl.BlockSpec((_TILE, d), lambda i: (i, 0)),
            pl.BlockSpec((_TILE, d), lambda i: (i, 0)),
            pl.BlockSpec((1, d), lambda i: (0, 0)),
            pl.BlockSpec((1, d), lambda i: (0, 0)),
        ],
        out_specs=[
            pl.BlockSpec((_TILE, d), lambda i: (i, 0)),
            pl.BlockSpec((_TILE, d), lambda i: (i, 0)),
        ],
        out_shape=[jax.ShapeDtypeStruct((n, d), jnp.float32)] * 2,
        compiler_params=pltpu.CompilerParams(
            dimension_semantics=("arbitrary",),
        ),
    )(scale_w, psir, psii, alpha2, pot_eff)
    return jnp.stack([out_r, out_i], axis=-1).reshape(b, s, d, 2)


# speedup vs baseline: 4.5225x; 1.0042x over previous
import jax
import jax.numpy as jnp
from jax.experimental import pallas as pl
from jax.experimental.pallas import tpu as pltpu

_DT = 0.1
_HALF_DT = _DT / 2.0
_CG_MAX_ITER = 20
_CG_TOL = 1e-06
_DILS = (5, 10, 20)
_TILE = 256


def _ham(v, w, pot_eff):
    D = v.shape[1]
    out = pot_eff * v
    for s, d in enumerate(_DILS):
        out = out - w[s] * (pltpu.roll(v, d, 1) + pltpu.roll(v, D - d, 1))
    return out


def _body(scale_ref, psir_ref, psii_ref, alpha_ref, pot_ref, outr_ref, outi_ref):
    pr = psir_ref[:]
    pi_ = psii_ref[:]
    inten = pr * pr + pi_ * pi_
    m = jnp.mean(inten, axis=1, keepdims=True)
    inten = inten / (m + 1e-08)
    phase = alpha_ref[:] * inten
    c = jnp.cos(phase)
    sn = jnp.sin(phase)
    rot_r = pr * c - pi_ * sn
    rot_i = pr * sn + pi_ * c

    w = (scale_ref[0], scale_ref[1], scale_ref[2])
    pot = pot_ref[:]

    rhs_r = rot_r + _HALF_DT * _ham(rot_i, w, pot)
    rhs_i = rot_i - _HALF_DT * _ham(rot_r, w, pot)

    def tokdot(ar, ai, br, bi):
        return jnp.sum(ar * br + ai * bi, axis=1, keepdims=True)

    r_r = rhs_r
    r_i = rhs_i
    p_r = r_r
    p_i = r_i
    x_r = jnp.zeros_like(r_r)
    x_i = jnp.zeros_like(r_i)
    rs_old = tokdot(r_r, r_i, r_r, r_i)

    def cg_iter(it, carry):
        x_r, x_i, r_r, r_i, p_r, p_i, rs_old = carry
        active = jnp.sqrt(rs_old) > _CG_TOL
        Ap_r = p_r - _HALF_DT * _ham(p_i, w, pot)
        Ap_i = p_i + _HALF_DT * _ham(p_r, w, pot)
        pAp = tokdot(p_r, p_i, Ap_r, Ap_i)
        a = jnp.where(active, rs_old / (pAp + 1e-12), 0.0)
        x_r = x_r + a * p_r
        x_i = x_i + a * p_i
        r_r = r_r - a * Ap_r
        r_i = r_i - a * Ap_i
        rs_new = tokdot(r_r, r_i, r_r, r_i)
        beta = jnp.where(active, rs_new / (rs_old + 1e-12), 0.0)
        p_r = r_r + beta * p_r
        p_i = r_i + beta * p_i
        rs_old = jnp.where(active, rs_new, rs_old)
        return (x_r, x_i, r_r, r_i, p_r, p_i, rs_old)

    carry = (x_r, x_i, r_r, r_i, p_r, p_i, rs_old)
    carry = jax.lax.fori_loop(0, _CG_MAX_ITER, cg_iter, carry)
    outr_ref[:] = carry[0]
    outi_ref[:] = carry[1]


def kernel(psi, alpha, scale_w, potential):
    b, s, d, _ = psi.shape
    n = b * s
    psir = psi[..., 0].reshape(n, d)
    psii = psi[..., 1].reshape(n, d)
    alpha2 = alpha.reshape(1, d)
    pot_eff = (potential + 2.0 * jnp.sum(scale_w)).reshape(1, d)

    grid = (n // _TILE,)
    out_r, out_i = pl.pallas_call(
        _body,
        grid=grid,
        in_specs=[
            pl.BlockSpec(memory_space=pltpu.SMEM),
            pl.BlockSpec((_TILE, d), lambda i: (i, 0)),
            pl.BlockSpec((_TILE, d), lambda i: (i, 0)),
            pl.BlockSpec((1, d), lambda i: (0, 0)),
            pl.BlockSpec((1, d), lambda i: (0, 0)),
        ],
        out_specs=[
            pl.BlockSpec((_TILE, d), lambda i: (i, 0)),
            pl.BlockSpec((_TILE, d), lambda i: (i, 0)),
        ],
        out_shape=[jax.ShapeDtypeStruct((n, d), jnp.float32)] * 2,
        compiler_params=pltpu.CompilerParams(
            dimension_semantics=("arbitrary",),
        ),
    )(scale_w, psir, psii, alpha2, pot_eff)
    return jnp.stack([out_r, out_i], axis=-1).reshape(b, s, d, 2)


# speedup vs baseline: 5.6951x; 1.2593x over previous
import jax
import jax.numpy as jnp
from jax.experimental import pallas as pl
from jax.experimental.pallas import tpu as pltpu

_DT = 0.1
_HALF_DT = _DT / 2.0
_CG_MAX_ITER = 20
_CG_TOL = 1e-06
_DILS = (5, 10, 20)
_HALO = 24
_TILE = 256


def _store_haloed(sref, v, D):
    sref[pl.ds(_HALO, D), :] = v
    sref[pl.ds(0, _HALO), :] = v[D - _HALO:, :]
    sref[pl.ds(_HALO + D, _HALO), :] = v[:_HALO, :]


def _ham_from_scratch(sref, v, w, pot_eff, D):
    out = pot_eff * v
    for s, d in enumerate(_DILS):
        plus = sref[pl.ds(_HALO - d, D), :]
        minus = sref[pl.ds(_HALO + d, D), :]
        out = out - w[s] * (plus + minus)
    return out


def _body(scale_ref, psir_ref, psii_ref, alpha_ref, pot_ref, outr_ref, outi_ref,
          sr_ref, si_ref):
    D = psir_ref.shape[0]
    pr = psir_ref[:]
    pi_ = psii_ref[:]
    inten = pr * pr + pi_ * pi_
    m = jnp.mean(inten, axis=0, keepdims=True)
    inten = inten / (m + 1e-08)
    phase = alpha_ref[:] * inten
    c = jnp.cos(phase)
    sn = jnp.sin(phase)
    rot_r = pr * c - pi_ * sn
    rot_i = pr * sn + pi_ * c

    w = (scale_ref[0], scale_ref[1], scale_ref[2])
    pot = pot_ref[:]

    _store_haloed(sr_ref, rot_r, D)
    _store_haloed(si_ref, rot_i, D)
    rhs_r = rot_r + _HALF_DT * _ham_from_scratch(si_ref, rot_i, w, pot, D)
    rhs_i = rot_i - _HALF_DT * _ham_from_scratch(sr_ref, rot_r, w, pot, D)

    def tokdot(ar, ai, br, bi):
        return jnp.sum(ar * br + ai * bi, axis=0, keepdims=True)

    r_r = rhs_r
    r_i = rhs_i
    x_r = jnp.zeros_like(r_r)
    x_i = jnp.zeros_like(r_i)
    rs_old = tokdot(r_r, r_i, r_r, r_i)
    _store_haloed(sr_ref, r_r, D)
    _store_haloed(si_ref, r_i, D)

    def cg_iter(it, carry):
        x_r, x_i, r_r, r_i, rs_old = carry
        active = jnp.sqrt(rs_old) > _CG_TOL
        p_r = sr_ref[pl.ds(_HALO, D), :]
        p_i = si_ref[pl.ds(_HALO, D), :]
        Ap_r = p_r - _HALF_DT * _ham_from_scratch(si_ref, p_i, w, pot, D)
        Ap_i = p_i + _HALF_DT * _ham_from_scratch(sr_ref, p_r, w, pot, D)
        pAp = tokdot(p_r, p_i, Ap_r, Ap_i)
        a = jnp.where(active, rs_old / (pAp + 1e-12), 0.0)
        x_r = x_r + a * p_r
        x_i = x_i + a * p_i
        r_r = r_r - a * Ap_r
        r_i = r_i - a * Ap_i
        rs_new = tokdot(r_r, r_i, r_r, r_i)
        beta = jnp.where(active, rs_new / (rs_old + 1e-12), 0.0)
        _store_haloed(sr_ref, r_r + beta * p_r, D)
        _store_haloed(si_ref, r_i + beta * p_i, D)
        rs_old = jnp.where(active, rs_new, rs_old)
        return (x_r, x_i, r_r, r_i, rs_old)

    carry = (x_r, x_i, r_r, r_i, rs_old)
    carry = jax.lax.fori_loop(0, _CG_MAX_ITER, cg_iter, carry)
    outr_ref[:] = carry[0]
    outi_ref[:] = carry[1]


def kernel(psi, alpha, scale_w, potential):
    b, s, d, _ = psi.shape
    n = b * s
    psir = psi[..., 0].reshape(n, d).T
    psii = psi[..., 1].reshape(n, d).T
    alpha2 = alpha.reshape(d, 1)
    pot_eff = (potential + 2.0 * jnp.sum(scale_w)).reshape(d, 1)

    grid = (n // _TILE,)
    out_r, out_i = pl.pallas_call(
        _body,
        grid=grid,
        in_specs=[
            pl.BlockSpec(memory_space=pltpu.SMEM),
            pl.BlockSpec((d, _TILE), lambda i: (0, i)),
            pl.BlockSpec((d, _TILE), lambda i: (0, i)),
            pl.BlockSpec((d, 1), lambda i: (0, 0)),
            pl.BlockSpec((d, 1), lambda i: (0, 0)),
        ],
        out_specs=[
            pl.BlockSpec((d, _TILE), lambda i: (0, i)),
            pl.BlockSpec((d, _TILE), lambda i: (0, i)),
        ],
        out_shape=[jax.ShapeDtypeStruct((d, n), jnp.float32)] * 2,
        scratch_shapes=[
            pltpu.VMEM((d + 2 * _HALO, _TILE), jnp.float32),
            pltpu.VMEM((d + 2 * _HALO, _TILE), jnp.float32),
        ],
        compiler_params=pltpu.CompilerParams(
            dimension_semantics=("arbitrary",),
        ),
    )(scale_w, psir, psii, alpha2, pot_eff)
    return jnp.stack([out_r.T, out_i.T], axis=-1).reshape(b, s, d, 2)


# speedup vs baseline: 6.5497x; 1.1501x over previous
import jax
import jax.numpy as jnp
from jax.experimental import pallas as pl
from jax.experimental.pallas import tpu as pltpu

_DT = 0.1
_HALF_DT = _DT / 2.0
_CG_MAX_ITER = 20
_CG_TOL = 1e-06
_DILS = (5, 10, 20)
_HALO = 24
_TILE = 512


def _store_haloed(sref, v, D):
    sref[pl.ds(_HALO, D), :] = v
    sref[pl.ds(0, _HALO), :] = v[D - _HALO:, :]
    sref[pl.ds(_HALO + D, _HALO), :] = v[:_HALO, :]


def _ham_from_scratch(sref, v, w, pot_eff, D):
    out = pot_eff * v
    for s, d in enumerate(_DILS):
        plus = sref[pl.ds(_HALO - d, D), :]
        minus = sref[pl.ds(_HALO + d, D), :]
        out = out - w[s] * (plus + minus)
    return out


def _body(scale_ref, psir_ref, psii_ref, alpha_ref, pot_ref, outr_ref, outi_ref,
          sr_ref, si_ref):
    D = psir_ref.shape[0]
    pr = psir_ref[:]
    pi_ = psii_ref[:]
    inten = pr * pr + pi_ * pi_
    m = jnp.mean(inten, axis=0, keepdims=True)
    inten = inten / (m + 1e-08)
    phase = alpha_ref[:] * inten
    c = jnp.cos(phase)
    sn = jnp.sin(phase)
    rot_r = pr * c - pi_ * sn
    rot_i = pr * sn + pi_ * c

    w = (scale_ref[0], scale_ref[1], scale_ref[2])
    pot = pot_ref[:]

    _store_haloed(sr_ref, rot_r, D)
    _store_haloed(si_ref, rot_i, D)
    rhs_r = rot_r + _HALF_DT * _ham_from_scratch(si_ref, rot_i, w, pot, D)
    rhs_i = rot_i - _HALF_DT * _ham_from_scratch(sr_ref, rot_r, w, pot, D)

    def tokdot(ar, ai, br, bi):
        return jnp.sum(ar * br + ai * bi, axis=0, keepdims=True)

    r_r = rhs_r
    r_i = rhs_i
    x_r = jnp.zeros_like(r_r)
    x_i = jnp.zeros_like(r_i)
    rs_old = tokdot(r_r, r_i, r_r, r_i)
    _store_haloed(sr_ref, r_r, D)
    _store_haloed(si_ref, r_i, D)

    def cg_iter(it, carry):
        x_r, x_i, r_r, r_i, rs_old = carry
        active = jnp.sqrt(rs_old) > _CG_TOL
        p_r = sr_ref[pl.ds(_HALO, D), :]
        p_i = si_ref[pl.ds(_HALO, D), :]
        Ap_r = p_r - _HALF_DT * _ham_from_scratch(si_ref, p_i, w, pot, D)
        Ap_i = p_i + _HALF_DT * _ham_from_scratch(sr_ref, p_r, w, pot, D)
        pAp = tokdot(p_r, p_i, Ap_r, Ap_i)
        a = jnp.where(active, rs_old / (pAp + 1e-12), 0.0)
        x_r = x_r + a * p_r
        x_i = x_i + a * p_i
        r_r = r_r - a * Ap_r
        r_i = r_i - a * Ap_i
        rs_new = tokdot(r_r, r_i, r_r, r_i)
        beta = jnp.where(active, rs_new / (rs_old + 1e-12), 0.0)
        _store_haloed(sr_ref, r_r + beta * p_r, D)
        _store_haloed(si_ref, r_i + beta * p_i, D)
        rs_old = jnp.where(active, rs_new, rs_old)
        return (x_r, x_i, r_r, r_i, rs_old)

    carry = (x_r, x_i, r_r, r_i, rs_old)
    carry = jax.lax.fori_loop(0, _CG_MAX_ITER, cg_iter, carry, unroll=2)
    outr_ref[:] = carry[0]
    outi_ref[:] = carry[1]


def kernel(psi, alpha, scale_w, potential):
    b, s, d, _ = psi.shape
    n = b * s
    psir = psi[..., 0].reshape(n, d).T
    psii = psi[..., 1].reshape(n, d).T
    alpha2 = alpha.reshape(d, 1)
    pot_eff = (potential + 2.0 * jnp.sum(scale_w)).reshape(d, 1)

    grid = (n // _TILE,)
    out_r, out_i = pl.pallas_call(
        _body,
        grid=grid,
        in_specs=[
            pl.BlockSpec(memory_space=pltpu.SMEM),
            pl.BlockSpec((d, _TILE), lambda i: (0, i)),
            pl.BlockSpec((d, _TILE), lambda i: (0, i)),
            pl.BlockSpec((d, 1), lambda i: (0, 0)),
            pl.BlockSpec((d, 1), lambda i: (0, 0)),
        ],
        out_specs=[
            pl.BlockSpec((d, _TILE), lambda i: (0, i)),
            pl.BlockSpec((d, _TILE), lambda i: (0, i)),
        ],
        out_shape=[jax.ShapeDtypeStruct((d, n), jnp.float32)] * 2,
        scratch_shapes=[
            pltpu.VMEM((d + 2 * _HALO, _TILE), jnp.float32),
            pltpu.VMEM((d + 2 * _HALO, _TILE), jnp.float32),
        ],
        compiler_params=pltpu.CompilerParams(
            dimension_semantics=("arbitrary",),
        ),
    )(scale_w, psir, psii, alpha2, pot_eff)
    return jnp.stack([out_r.T, out_i.T], axis=-1).reshape(b, s, d, 2)


# speedup vs baseline: 6.8524x; 1.0462x over previous
import jax
import jax.numpy as jnp
from jax.experimental import pallas as pl
from jax.experimental.pallas import tpu as pltpu

_DT = 0.1
_HALF_DT = _DT / 2.0
_CG_MAX_ITER = 20
_CG_TOL = 1e-06
_DILS = (5, 10, 20)
_HALO = 24
_TILE = 512


def _store_haloed(sref, v, D):
    sref[pl.ds(_HALO, D), :] = v
    sref[pl.ds(0, _HALO), :] = v[D - _HALO:, :]
    sref[pl.ds(_HALO + D, _HALO), :] = v[:_HALO, :]


def _ham_from_scratch(sref, v, w, pot_eff, D):
    out = pot_eff * v
    for s, d in enumerate(_DILS):
        plus = sref[pl.ds(_HALO - d, D), :]
        minus = sref[pl.ds(_HALO + d, D), :]
        out = out - w[s] * (plus + minus)
    return out


def _body(scale_ref, psir_ref, psii_ref, alpha_ref, pot_ref, outr_ref, outi_ref,
          sr_ref, si_ref):
    D = psir_ref.shape[0]
    pr = psir_ref[:]
    pi_ = psii_ref[:]
    inten = pr * pr + pi_ * pi_
    m = jnp.mean(inten, axis=0, keepdims=True)
    inten = inten / (m + 1e-08)
    phase = alpha_ref[:] * inten
    c = jnp.cos(phase)
    sn = jnp.sin(phase)
    rot_r = pr * c - pi_ * sn
    rot_i = pr * sn + pi_ * c

    w = (scale_ref[0], scale_ref[1], scale_ref[2])
    pot = pot_ref[:]

    _store_haloed(sr_ref, rot_r, D)
    _store_haloed(si_ref, rot_i, D)
    rhs_r = rot_r + _HALF_DT * _ham_from_scratch(si_ref, rot_i, w, pot, D)
    rhs_i = rot_i - _HALF_DT * _ham_from_scratch(sr_ref, rot_r, w, pot, D)

    def tokdot(ar, ai, br, bi):
        return jnp.sum(ar * br + ai * bi, axis=0, keepdims=True)

    r_r = rhs_r
    r_i = rhs_i
    outr_ref[:] = jnp.zeros_like(r_r)
    outi_ref[:] = jnp.zeros_like(r_i)
    rs_old = tokdot(r_r, r_i, r_r, r_i)
    _store_haloed(sr_ref, r_r, D)
    _store_haloed(si_ref, r_i, D)

    def cg_iter(it, carry):
        r_r, r_i, rs_old = carry
        active = jnp.sqrt(rs_old) > _CG_TOL
        p_r = sr_ref[pl.ds(_HALO, D), :]
        p_i = si_ref[pl.ds(_HALO, D), :]
        Ap_r = p_r - _HALF_DT * _ham_from_scratch(si_ref, p_i, w, pot, D)
        Ap_i = p_i + _HALF_DT * _ham_from_scratch(sr_ref, p_r, w, pot, D)
        pAp = tokdot(p_r, p_i, Ap_r, Ap_i)
        a = jnp.where(active, rs_old / (pAp + 1e-12), 0.0)
        outr_ref[:] += a * p_r
        outi_ref[:] += a * p_i
        r_r = r_r - a * Ap_r
        r_i = r_i - a * Ap_i
        rs_new = tokdot(r_r, r_i, r_r, r_i)
        beta = jnp.where(active, rs_new / (rs_old + 1e-12), 0.0)
        _store_haloed(sr_ref, r_r + beta * p_r, D)
        _store_haloed(si_ref, r_i + beta * p_i, D)
        rs_old = jnp.where(active, rs_new, rs_old)
        return (r_r, r_i, rs_old)

    carry = (r_r, r_i, rs_old)
    carry = jax.lax.fori_loop(0, _CG_MAX_ITER, cg_iter, carry, unroll=2)


def kernel(psi, alpha, scale_w, potential):
    b, s, d, _ = psi.shape
    n = b * s
    psir = psi[..., 0].reshape(n, d).T
    psii = psi[..., 1].reshape(n, d).T
    alpha2 = alpha.reshape(d, 1)
    pot_eff = (potential + 2.0 * jnp.sum(scale_w)).reshape(d, 1)

    grid = (n // _TILE,)
    out_r, out_i = pl.pallas_call(
        _body,
        grid=grid,
        in_specs=[
            pl.BlockSpec(memory_space=pltpu.SMEM),
            pl.BlockSpec((d, _TILE), lambda i: (0, i)),
            pl.BlockSpec((d, _TILE), lambda i: (0, i)),
            pl.BlockSpec((d, 1), lambda i: (0, 0)),
            pl.BlockSpec((d, 1), lambda i: (0, 0)),
        ],
        out_specs=[
            pl.BlockSpec((d, _TILE), lambda i: (0, i)),
            pl.BlockSpec((d, _TILE), lambda i: (0, i)),
        ],
        out_shape=[jax.ShapeDtypeStruct((d, n), jnp.float32)] * 2,
        scratch_shapes=[
            pltpu.VMEM((d + 2 * _HALO, _TILE), jnp.float32),
            pltpu.VMEM((d + 2 * _HALO, _TILE), jnp.float32),
        ],
        compiler_params=pltpu.CompilerParams(
            dimension_semantics=("arbitrary",),
        ),
    )(scale_w, psir, psii, alpha2, pot_eff)
    return jnp.stack([out_r.T, out_i.T], axis=-1).reshape(b, s, d, 2)
